# Initial kernel scaffold; baseline (speedup 1.0000x reference)
import functools

import jax
import jax.numpy as jnp
from jax import lax
from jax.experimental import pallas as pl
from jax.experimental.pallas import tpu as pltpu
from jax.experimental.pallas import tpu_sc as plsc

D = 10000
E = 320000
F = 128


BK = 1000
BD = 1000


def _mm_body(x_ref, w_ref, o_ref):
    k = pl.program_id(1)

    @pl.when(k == 0)
    def _():
        o_ref[...] = jnp.zeros_like(o_ref)

    o_ref[...] += lax.dot_general(
        w_ref[...], x_ref[...],
        (((0,), (1,)), ((), ())),
        preferred_element_type=jnp.float32,
    )


def _matmul_support_t(x, w):
    return pl.pallas_call(
        _mm_body,
        grid=(D // BD, D // BK),
        in_specs=[
            pl.BlockSpec((F, BK), lambda d, k: (0, k)),
            pl.BlockSpec((BK, BD), lambda d, k: (k, d)),
        ],
        out_specs=pl.BlockSpec((BD, F), lambda d, k: (d, 0)),
        out_shape=jax.ShapeDtypeStruct((D, F), jnp.float32),
        compiler_params=pltpu.CompilerParams(
            dimension_semantics=("parallel", "arbitrary"),
        ),
    )(x, w)


NC = 2
NS = 16
C = 80
EDGES_PER_TILE = E // (NC * NS)
NCHUNK = EDGES_PER_TILE // C
ROWS_PER_TILE = D // NS
WB = 125


def _spmm_body(sup_hbm, rows_hbm, cols_hbm, vals_hbm, out_hbm,
               colv, rowv, valv, gath, stage, acc, sem):
    c = lax.axis_index("c")
    s = lax.axis_index("s")

    zeros16 = jnp.zeros((16,), jnp.float32)

    def _zrow(r, carry):
        for k in range(F // 16):
            stage[r, pl.ds(k * 16, 16)] = zeros16
        return carry

    lax.fori_loop(0, WB, _zrow, 0)
    for i in range(ROWS_PER_TILE // WB):
        pltpu.sync_copy(stage, acc.at[pl.ds(s * ROWS_PER_TILE + i * WB, WB)])
    plsc.subcore_barrier()

    base = c * (E // NC) + s * EDGES_PER_TILE

    def _chunk(ch, carry):
        eb = base + ch * C
        pltpu.sync_copy(cols_hbm.at[pl.ds(eb, C)], colv)
        pltpu.sync_copy(rows_hbm.at[pl.ds(eb, C)], rowv)
        pltpu.sync_copy(vals_hbm.at[pl.ds(eb, C)], valv)
        pltpu.async_copy(sup_hbm.at[colv], gath, sem).wait()

        def _grp(g, cc):
            for l in range(16):
                j = g * 16 + l
                bc = plsc.load_gather(valv, [jnp.full((16,), j, jnp.int32)])
                for k in range(F // 16):
                    sl = pl.ds(k * 16, 16)
                    gath[j, sl] = gath[j, sl] * bc
            return cc

        lax.fori_loop(0, C // 16, _grp, 0)
        pltpu.sync_copy(gath, acc.at[rowv], add=True)
        return carry

    lax.fori_loop(0, NCHUNK, _chunk, 0)
    plsc.subcore_barrier()

    for i in range(ROWS_PER_TILE // WB):
        r0 = s * ROWS_PER_TILE + i * WB
        pltpu.sync_copy(acc.at[pl.ds(r0, WB)], stage)
        pltpu.sync_copy(stage, out_hbm.at[c, pl.ds(r0, WB)])


def _spmm(sup_t, rows, cols, vals):
    mesh = plsc.VectorSubcoreMesh(core_axis_name="c", subcore_axis_name="s")
    fn = functools.partial(
        pl.kernel,
        mesh=mesh,
        out_type=jax.ShapeDtypeStruct((NC, D, F), jnp.float32),
        scratch_types=[
            pltpu.VMEM((C,), jnp.int32),
            pltpu.VMEM((C,), jnp.int32),
            pltpu.VMEM((C,), jnp.float32),
            pltpu.VMEM((C, F), jnp.float32),
            pltpu.VMEM((WB, F), jnp.float32),
            pltpu.VMEM_SHARED((D, F), jnp.float32),
            pltpu.SemaphoreType.DMA,
        ],
    )(_spmm_body)
    return fn(sup_t, rows, cols, vals)


CB = 1000


def _comb_body(a_ref, b_ref, bias_ref, o_ref):
    st = a_ref[...] + b_ref[...]
    o_ref[...] = st.T + bias_ref[...]


def _combine(p0, p1, bias):
    return pl.pallas_call(
        _comb_body,
        grid=(D // CB,),
        in_specs=[
            pl.BlockSpec((CB, F), lambda i: (i, 0)),
            pl.BlockSpec((CB, F), lambda i: (i, 0)),
            pl.BlockSpec((1, CB), lambda i: (0, i)),
        ],
        out_specs=pl.BlockSpec((F, CB), lambda i: (0, i)),
        out_shape=jax.ShapeDtypeStruct((F, D), jnp.float32),
    )(p0, p1, bias)


def kernel(input, adj_indices, adj_values, weight, bias):
    rows = adj_indices[0]
    cols = adj_indices[1]
    sup_t = _matmul_support_t(input, weight)
    partials = _spmm(sup_t, rows, cols, adj_values)
    return _combine(partials[0], partials[1], bias.reshape(1, D))


# speedup vs baseline: 3.6659x; 3.6659x over previous
import functools

import jax
import jax.numpy as jnp
from jax import lax
from jax.experimental import pallas as pl
from jax.experimental.pallas import tpu as pltpu
from jax.experimental.pallas import tpu_sc as plsc

D = 10000
E = 320000
F = 128


BK = 400


def _mm_body(xt_ref, w_ref, o_ref):
    k = pl.program_id(0)

    @pl.when(k == 0)
    def _():
        o_ref[...] = jnp.zeros_like(o_ref)

    o_ref[...] += lax.dot_general(
        w_ref[...], xt_ref[...],
        (((0,), (0,)), ((), ())),
        preferred_element_type=jnp.float32,
    )


def _matmul_support_t(x_t, w):
    return pl.pallas_call(
        _mm_body,
        grid=(D // BK,),
        in_specs=[
            pl.BlockSpec((BK, F), lambda k: (k, 0)),
            pl.BlockSpec((BK, D), lambda k: (k, 0)),
        ],
        out_specs=pl.BlockSpec((D, F), lambda k: (0, 0)),
        out_shape=jax.ShapeDtypeStruct((D, F), jnp.float32),
        compiler_params=pltpu.CompilerParams(
            dimension_semantics=("arbitrary",),
        ),
    )(x_t, w)


NC = 2
NS = 16
C = 80
EDGES_PER_TILE = E // (NC * NS)
NCHUNK = EDGES_PER_TILE // C
TROWS = 624
WB = 208
REM0 = NS * TROWS
REM = D - REM0


def _spmm_body(sup_hbm, rows_hbm, cols_hbm, vals_hbm, out_hbm,
               colv, rowv, valv, gath, stage, acc, sem):
    c = lax.axis_index("c")
    s = lax.axis_index("s")

    zeros16 = jnp.zeros((16,), jnp.float32)

    def _zrow(r, carry):
        for k in range(F // 16):
            stage[r, pl.ds(k * 16, 16)] = zeros16
        return carry

    lax.fori_loop(0, WB, _zrow, 0)
    for i in range(TROWS // WB):
        pltpu.sync_copy(stage, acc.at[pl.ds(s * TROWS + i * WB, WB)])

    @pl.when(s == 0)
    def _():
        pltpu.sync_copy(stage.at[pl.ds(0, REM)], acc.at[pl.ds(REM0, REM)])

    plsc.subcore_barrier()

    base = c * (E // NC) + s * EDGES_PER_TILE

    def _chunk(ch, carry):
        eb = base + ch * C
        pltpu.sync_copy(cols_hbm.at[pl.ds(eb, C)], colv)
        pltpu.sync_copy(rows_hbm.at[pl.ds(eb, C)], rowv)
        pltpu.sync_copy(vals_hbm.at[pl.ds(eb, C)], valv)
        pltpu.async_copy(sup_hbm.at[colv], gath, sem).wait()

        def _grp(g, cc):
            vv = valv[pl.ds(g * 16, 16)]
            for l in range(16):
                j = g * 16 + l
                bc = jnp.full((16,), vv[l], jnp.float32)
                for k in range(F // 16):
                    sl = pl.ds(k * 16, 16)
                    gath[j, sl] = gath[j, sl] * bc
            return cc

        lax.fori_loop(0, C // 16, _grp, 0)
        pltpu.sync_copy(gath, acc.at[rowv], add=True)
        return carry

    lax.fori_loop(0, NCHUNK, _chunk, 0)
    plsc.subcore_barrier()

    for i in range(TROWS // WB):
        r0 = s * TROWS + i * WB
        pltpu.sync_copy(acc.at[pl.ds(r0, WB)], stage)
        pltpu.sync_copy(stage, out_hbm.at[c, pl.ds(r0, WB)])

    @pl.when(s == 0)
    def _():
        pltpu.sync_copy(acc.at[pl.ds(REM0, REM)], stage.at[pl.ds(0, REM)])
        pltpu.sync_copy(stage.at[pl.ds(0, REM)], out_hbm.at[c, pl.ds(REM0, REM)])


def _spmm(sup_t, rows, cols, vals):
    mesh = plsc.VectorSubcoreMesh(core_axis_name="c", subcore_axis_name="s")
    fn = functools.partial(
        pl.kernel,
        mesh=mesh,
        out_type=jax.ShapeDtypeStruct((NC, D, F), jnp.float32),
        scratch_types=[
            pltpu.VMEM((C,), jnp.int32),
            pltpu.VMEM((C,), jnp.int32),
            pltpu.VMEM((C,), jnp.float32),
            pltpu.VMEM((C, F), jnp.float32),
            pltpu.VMEM((WB, F), jnp.float32),
            pltpu.VMEM_SHARED((D, F), jnp.float32),
            pltpu.SemaphoreType.DMA,
        ],
    )(_spmm_body)
    return fn(sup_t, rows, cols, vals)


def _comb_body(a_ref, b_ref, bias_ref, o_ref):
    st = a_ref[...] + b_ref[...]
    o_ref[...] = st.T + bias_ref[...]


def _combine(p0, p1, bias):
    return pl.pallas_call(
        _comb_body,
        out_shape=jax.ShapeDtypeStruct((F, D), jnp.float32),
    )(p0, p1, bias)


def kernel(input, adj_indices, adj_values, weight, bias):
    rows = adj_indices[0]
    cols = adj_indices[1]
    sup_t = _matmul_support_t(input.T, weight)
    partials = _spmm(sup_t, rows, cols, adj_values)
    return _combine(partials[0], partials[1], bias.reshape(1, D))
